# baseline (speedup 1.0000x reference)
import functools
import math

import jax
import jax.numpy as jnp
from jax import lax
from jax.experimental import pallas as pl
from jax.experimental.pallas import tpu as pltpu
from jax.experimental.pallas import tpu_sc as plsc

D_MODEL = 64
SCALE = math.sqrt(D_MODEL)

NC = 2
NS = 16
NW = NC * NS

GSIZE = 128
GPC = 4
CHUNK = GPC * GSIZE


@functools.cache
def _build(n_blk):
    n_rows = n_blk * GSIZE
    n_chunks = n_rows // CHUNK
    total = NW * n_rows

    def body(table_hbm, idx_hbm, out_hbm, idx_v, rows_v, gsem):
        wid = lax.axis_index("s") * NC + lax.axis_index("c")
        base = wid * n_rows
        pltpu.sync_copy(idx_hbm.at[wid], idx_v)

        def chunk_body(c, carry):
            waits = []
            for j in range(GPC):
                waits.append(pltpu.async_copy(
                    table_hbm.at[idx_v.at[c * GPC + j]],
                    rows_v.at[pl.ds(j * GSIZE, GSIZE)], gsem))
            for w in waits:
                w.wait()

            def scale_row(r, carry2):
                for col in range(D_MODEL // 16):
                    sl = pl.ds(col * 16, 16)
                    rows_v[r, sl] = rows_v[r, sl] * SCALE
                return carry2
            lax.fori_loop(0, CHUNK, scale_row, 0)

            pltpu.sync_copy(rows_v, out_hbm.at[pl.ds(base + c * CHUNK, CHUNK)])
            return carry

        lax.fori_loop(0, n_chunks, chunk_body, 0)

    return pl.kernel(
        body,
        mesh=plsc.VectorSubcoreMesh(core_axis_name="c", subcore_axis_name="s"),
        compiler_params=pltpu.CompilerParams(use_tc_tiling_on_sc=False),
        out_type=jax.ShapeDtypeStruct((total, D_MODEL), jnp.float32),
        scratch_types=[
            pltpu.VMEM((n_blk, GSIZE), jnp.int32),
            pltpu.VMEM((CHUNK, D_MODEL), jnp.float32),
            pltpu.SemaphoreType.DMA,
        ],
    )


def kernel(x, table):
    orig_shape = x.shape
    flat = x.reshape(-1).astype(jnp.int32)
    n = flat.shape[0]
    grain = NW * GSIZE * GPC
    pad = (-n) % grain
    if pad:
        flat = jnp.concatenate([flat, jnp.zeros((pad,), jnp.int32)])
    n_blk = flat.shape[0] // (NW * GSIZE)
    idx3 = flat.reshape(NW, n_blk, GSIZE)
    out = _build(n_blk)(table, idx3)
    if pad:
        out = out[:n]
    return out.reshape(*orig_shape, D_MODEL)


# speedup vs baseline: 1.1160x; 1.1160x over previous
import functools
import math

import jax
import jax.numpy as jnp
from jax import lax
from jax.experimental import pallas as pl
from jax.experimental.pallas import tpu as pltpu
from jax.experimental.pallas import tpu_sc as plsc

D_MODEL = 64
SCALE = math.sqrt(D_MODEL)

NC = 2
NS = 16
NW = NC * NS

GSIZE = 128
GPC = 2
CHUNK = GPC * GSIZE
NBUF = 4
PREF = 2


@functools.cache
def _build(n_blk):
    n_rows = n_blk * GSIZE
    n_chunks = n_rows // CHUNK
    total = NW * n_rows
    assert n_chunks % NBUF == 0 and n_chunks >= NBUF

    def body(table_hbm, idx_hbm, out_hbm, idx_v, rows, gsems, wsems):
        wid = lax.axis_index("s") * NC + lax.axis_index("c")
        base = wid * n_rows
        pltpu.sync_copy(idx_hbm.at[wid], idx_v)

        def fire_gather(c, b):
            for j in range(GPC):
                pltpu.async_copy(
                    table_hbm.at[idx_v.at[c * GPC + j]],
                    rows.at[b, pl.ds(j * GSIZE, GSIZE)], gsems.at[b])

        def wait_gather(b):
            pltpu.make_async_copy(
                table_hbm.at[pl.ds(0, CHUNK)], rows.at[b], gsems.at[b]).wait()

        def fire_write(c, b):
            pltpu.async_copy(
                rows.at[b], out_hbm.at[pl.ds(base + c * CHUNK, CHUNK)],
                wsems.at[b])

        def wait_write(b):
            pltpu.make_async_copy(
                rows.at[b], out_hbm.at[pl.ds(base, CHUNK)], wsems.at[b]).wait()

        for b in range(PREF):
            fire_gather(b, b)

        def pipe_body(p, carry):
            for b in range(NBUF):
                c = p * NBUF + b
                wait_gather(b)

                @plsc.parallel_loop(0, CHUNK, unroll=16)
                def _(r):
                    for col in range(D_MODEL // 16):
                        sl = pl.ds(col * 16, 16)
                        rows[b, r, sl] = rows[b, r, sl] * SCALE

                fire_write(c, b)
                bf = (b + PREF) % NBUF

                @pl.when(c >= NBUF - PREF)
                def _():
                    wait_write(bf)

                @pl.when(c + PREF < n_chunks)
                def _():
                    fire_gather(c + PREF, bf)
            return carry

        lax.fori_loop(0, n_chunks // NBUF, pipe_body, 0)

        for b in range(NBUF - PREF, NBUF):
            wait_write(b)

    return pl.kernel(
        body,
        mesh=plsc.VectorSubcoreMesh(core_axis_name="c", subcore_axis_name="s"),
        compiler_params=pltpu.CompilerParams(use_tc_tiling_on_sc=False),
        out_type=jax.ShapeDtypeStruct((total, D_MODEL), jnp.float32),
        scratch_types=[
            pltpu.VMEM((n_blk, GSIZE), jnp.int32),
            pltpu.VMEM((NBUF, CHUNK, D_MODEL), jnp.float32),
            pltpu.SemaphoreType.DMA((NBUF,)),
            pltpu.SemaphoreType.DMA((NBUF,)),
        ],
    )


def kernel(x, table):
    orig_shape = x.shape
    flat = x.reshape(-1).astype(jnp.int32)
    n = flat.shape[0]
    grain = NW * CHUNK * NBUF
    pad = (-n) % grain
    if pad:
        flat = jnp.concatenate([flat, jnp.zeros((pad,), jnp.int32)])
    n_blk = flat.shape[0] // (NW * GSIZE)
    idx3 = flat.reshape(NW, n_blk, GSIZE)
    out = _build(n_blk)(table, idx3)
    if pad:
        out = out[:n]
    return out.reshape(*orig_shape, D_MODEL)


# speedup vs baseline: 1.9794x; 1.7737x over previous
import functools
import math

import jax
import jax.numpy as jnp
from jax import lax
from jax.experimental import pallas as pl
from jax.experimental.pallas import tpu as pltpu
from jax.experimental.pallas import tpu_sc as plsc

D_MODEL = 64
SCALE = math.sqrt(D_MODEL)

NC = 2
NS = 16
NW = NC * NS

GSIZE = 128
NBUF = 4
PREF = 2


@functools.cache
def _build(n_b):
    def body(table_hbm, idx_hbm, out_hbm, idx_v, gbuf, obuf, gsems, wsems):
        wid = lax.axis_index("s") * NC + lax.axis_index("c")
        jota = lax.iota(jnp.int32, 16)
        pltpu.sync_copy(idx_hbm.at[wid], idx_v)

        def fire_gather(c, b):
            pltpu.async_copy(
                table_hbm.at[idx_v.at[c]], gbuf.at[b], gsems.at[b])

        def wait_gather(b):
            pltpu.make_async_copy(
                table_hbm.at[pl.ds(0, GSIZE)], gbuf.at[b], gsems.at[b]).wait()

        def fire_write(c, b):
            pltpu.async_copy(
                obuf.at[b, :, :, :, pl.ds(0, GSIZE)],
                out_hbm.at[c, pl.ds(0, D_MODEL // 8), pl.ds(wid, 1)],
                wsems.at[b])

        def wait_write(b):
            pltpu.make_async_copy(
                obuf.at[b, :, :, :, pl.ds(0, GSIZE)],
                out_hbm.at[0, pl.ds(0, D_MODEL // 8), pl.ds(0, 1)],
                wsems.at[b]).wait()

        zv = 0 * jota
        jhis = [(j0 * 16 + jota) >> 3 for j0 in range(D_MODEL // 16)]
        jlos = [(j0 * 16 + jota) & 7 for j0 in range(D_MODEL // 16)]

        def transpose_block(b):
            @plsc.parallel_loop(0, GSIZE, unroll=8)
            def _(a):
                av = a + zv
                for j0 in range(D_MODEL // 16):
                    vals = gbuf[b, a, pl.ds(j0 * 16, 16)]
                    plsc.store_scatter(
                        obuf.at[b], [jhis[j0], zv, jlos[j0], av],
                        vals * SCALE)

        for b in range(PREF):
            fire_gather(b, b)

        def pipe_body(p, carry):
            for b in range(NBUF):
                c = p * NBUF + b
                wait_gather(b)
                bf = (b + PREF) % NBUF

                @pl.when(c >= NBUF - PREF)
                def _():
                    wait_write(bf)

                @pl.when(c + PREF < n_b)
                def _():
                    fire_gather(c + PREF, bf)

                transpose_block(b)
                fire_write(c, b)
            return carry

        lax.fori_loop(0, n_b // NBUF, pipe_body, 0)

        for b in range(NBUF - PREF, NBUF):
            wait_write(b)

    return pl.kernel(
        body,
        mesh=plsc.VectorSubcoreMesh(core_axis_name="c", subcore_axis_name="s"),
        compiler_params=pltpu.CompilerParams(
            use_tc_tiling_on_sc=False, needs_layout_passes=False),
        out_type=jax.ShapeDtypeStruct(
            (n_b, D_MODEL // 8, NW, 8, GSIZE), jnp.float32),
        scratch_types=[
            pltpu.VMEM((n_b, GSIZE), jnp.int32),
            pltpu.VMEM((NBUF, GSIZE, D_MODEL), jnp.float32),
            pltpu.VMEM((NBUF, D_MODEL // 8, 1, 8, 136), jnp.float32),
            pltpu.SemaphoreType.DMA((NBUF,)),
            pltpu.SemaphoreType.DMA((NBUF,)),
        ],
    )


def kernel(x, table):
    a_dim, b_dim = x.shape
    assert a_dim == NW * GSIZE and b_dim % NBUF == 0
    tpad = jnp.pad(table, ((0, 0), (0, 2 * D_MODEL - table.shape[1])))
    tlin = tpad.reshape(2 * table.shape[0], D_MODEL)
    idx_t = (x.astype(jnp.int32) * 2).T.reshape(
        b_dim, NW, GSIZE).transpose(1, 0, 2)
    o5 = _build(b_dim)(tlin, idx_t)
    return o5.transpose(2, 4, 0, 1, 3).reshape(a_dim, b_dim, D_MODEL)
